# Initial kernel scaffold; baseline (speedup 1.0000x reference)
import jax
import jax.numpy as jnp
from jax.experimental import pallas as pl


def kernel(x, edge_index, graph_ids, W_lift, b_lift, W_ro, b_ro, W_msg0, b_msg0, W_out0, b_out0, W_msg1, b_msg1, W_out1, b_out1, W_msg2, b_msg2, W_out2, b_out2):
    raise NotImplementedError("write your pallas kernel here")


# speedup vs baseline: 3.3445x; 3.3445x over previous
import functools

import jax
import jax.numpy as jnp
from jax import lax
from jax.experimental import pallas as pl
from jax.experimental.pallas import tpu as pltpu
from jax.experimental.pallas import tpu_sc as plsc

_NN = 10000
_EE = 160000
_RAW = 119
_RAWP = 128
_H = 300
_HP = 320
_HH = 160
_B = 10
_C = 2

_BLK = 1000
_GRID = _NN // _BLK

_NTILES = 16
_CHUNK = 80
_CPG = 16
_NGRP = 8
_EPT = _CHUNK * _CPG * _NGRP
_EPAD = _EPT * _NTILES
_ACC_ROWS = 10112
_ZROWS = _ACC_ROWS // _NTILES
_OROWS = 624


def _lift_msg_body(x_ref, wl_ref, bl_ref, wm_ref, bm_ref, ma_ref, mb_ref):
    h = jnp.dot(x_ref[...], wl_ref[...], preferred_element_type=jnp.float32)
    h = h + bl_ref[...]
    m = jnp.dot(h, wm_ref[...], preferred_element_type=jnp.float32) + bm_ref[...]
    m = jnp.maximum(m, 0.0)
    ma_ref[...] = m[:, :_HH]
    mb_ref[...] = m[:, _HH:]


def _lift_msg(xp, wl, bl, wm, bm):
    return pl.pallas_call(
        _lift_msg_body,
        grid=(_GRID,),
        in_specs=[
            pl.BlockSpec((_BLK, _RAWP), lambda i: (i, 0)),
            pl.BlockSpec((_RAWP, _HP), lambda i: (0, 0)),
            pl.BlockSpec((1, _HP), lambda i: (0, 0)),
            pl.BlockSpec((_HP, _HP), lambda i: (0, 0)),
            pl.BlockSpec((1, _HP), lambda i: (0, 0)),
        ],
        out_specs=[pl.BlockSpec((_BLK, _HH), lambda i: (i, 0)),
                   pl.BlockSpec((_BLK, _HH), lambda i: (i, 0))],
        out_shape=[jax.ShapeDtypeStruct((_NN, _HH), jnp.float32),
                   jax.ShapeDtypeStruct((_NN, _HH), jnp.float32)],
    )(xp, wl, bl, wm, bm)


def _hidden(aa_ref, ab_ref, wo_ref, bo_ref):
    h = jnp.dot(aa_ref[...], wo_ref[0:_HH, :], preferred_element_type=jnp.float32)
    h = h + jnp.dot(ab_ref[...], wo_ref[_HH:_HP, :], preferred_element_type=jnp.float32)
    return jnp.maximum(h + bo_ref[...], 0.0)


def _out_msg_body(aa_ref, ab_ref, wo_ref, bo_ref, wm_ref, bm_ref, ma_ref, mb_ref):
    h = _hidden(aa_ref, ab_ref, wo_ref, bo_ref)
    m = jnp.dot(h, wm_ref[...], preferred_element_type=jnp.float32) + bm_ref[...]
    m = jnp.maximum(m, 0.0)
    ma_ref[...] = m[:, :_HH]
    mb_ref[...] = m[:, _HH:]


def _out_msg(aa, ab, wo, bo, wm, bm):
    return pl.pallas_call(
        _out_msg_body,
        grid=(_GRID,),
        in_specs=[
            pl.BlockSpec((_BLK, _HH), lambda i: (i, 0)),
            pl.BlockSpec((_BLK, _HH), lambda i: (i, 0)),
            pl.BlockSpec((_HP, _HP), lambda i: (0, 0)),
            pl.BlockSpec((1, _HP), lambda i: (0, 0)),
            pl.BlockSpec((_HP, _HP), lambda i: (0, 0)),
            pl.BlockSpec((1, _HP), lambda i: (0, 0)),
        ],
        out_specs=[pl.BlockSpec((_BLK, _HH), lambda i: (i, 0)),
                   pl.BlockSpec((_BLK, _HH), lambda i: (i, 0))],
        out_shape=[jax.ShapeDtypeStruct((_NN, _HH), jnp.float32),
                   jax.ShapeDtypeStruct((_NN, _HH), jnp.float32)],
    )(aa, ab, wo, bo, wm, bm)


def _out_readout_body(aa_ref, ab_ref, wo_ref, bo_ref, gid_ref, wro_ref,
                      out_ref, acc_ref):
    i = pl.program_id(0)

    @pl.when(i == 0)
    def _():
        acc_ref[...] = jnp.zeros_like(acc_ref)

    h = _hidden(aa_ref, ab_ref, wo_ref, bo_ref)
    col = lax.broadcasted_iota(jnp.int32, (_BLK, _HP), 1)
    h = jnp.where(col == _HP - 1, 1.0, h)
    gid = gid_ref[0, 0, :]
    onehot = (lax.broadcasted_iota(jnp.int32, (16, _BLK), 0)
              == gid[None, :]).astype(jnp.float32)
    acc_ref[...] += jnp.dot(onehot, h, preferred_element_type=jnp.float32)

    @pl.when(i == _GRID - 1)
    def _():
        out_ref[...] = jnp.dot(acc_ref[...], wro_ref[...],
                               preferred_element_type=jnp.float32)


def _out_readout(aa, ab, wo, bo, gid3, wro):
    return pl.pallas_call(
        _out_readout_body,
        grid=(_GRID,),
        in_specs=[
            pl.BlockSpec((_BLK, _HH), lambda i: (i, 0)),
            pl.BlockSpec((_BLK, _HH), lambda i: (i, 0)),
            pl.BlockSpec((_HP, _HP), lambda i: (0, 0)),
            pl.BlockSpec((1, _HP), lambda i: (0, 0)),
            pl.BlockSpec((1, 1, _BLK), lambda i: (i, 0, 0)),
            pl.BlockSpec((_HP, 128), lambda i: (0, 0)),
        ],
        out_specs=pl.BlockSpec((16, 128), lambda i: (0, 0)),
        out_shape=jax.ShapeDtypeStruct((16, 128), jnp.float32),
        scratch_shapes=[pltpu.VMEM((16, _HP), jnp.float32)],
    )(aa, ab, wo, bo, gid3, wro)


def _sc_body(src_hbm, dst_hbm, ma_hbm, mb_hbm, oa_hbm, ob_hbm,
             acc, sidx, didx, b0, b1, s0, s1):
    cid = lax.axis_index("c")
    tid = lax.axis_index("s")
    bufs = (b0, b1)
    sems = (s0, s1)

    zero = jnp.zeros((16,), jnp.float32)

    def _zrow(i, carry):
        for j in range(_HH // 16):
            b0[i, pl.ds(j * 16, 16)] = zero
        return carry

    lax.fori_loop(0, _CHUNK, _zrow, 0)
    zb = tid * _ZROWS
    for q in range(_ZROWS // _CHUNK):
        pltpu.sync_copy(b0, acc.at[pl.ds(zb + q * _CHUNK, _CHUNK)])
    rem = _ZROWS % _CHUNK
    pltpu.sync_copy(b0.at[pl.ds(0, rem)],
                    acc.at[pl.ds(zb + _ZROWS - rem, rem)])
    plsc.subcore_barrier()

    def _run(m_hbm, o_hbm):
        def _group(g, carry):
            pltpu.sync_copy(src_hbm.at[tid, g], sidx)
            pltpu.sync_copy(dst_hbm.at[tid, g], didx)
            for b in range(2):
                pltpu.async_copy(m_hbm.at[sidx.at[b]], bufs[b], sems[b])
            for k in range(_CPG):
                b = k % 2
                pltpu.make_async_copy(m_hbm.at[sidx.at[k]], bufs[b],
                                      sems[b]).wait()
                pltpu.sync_copy(bufs[b], acc.at[didx.at[k]], add=True)
                if k + 2 < _CPG:
                    pltpu.async_copy(m_hbm.at[sidx.at[k + 2]], bufs[b],
                                     sems[b])
            return carry

        lax.fori_loop(0, _NGRP, _group, 0)
        plsc.subcore_barrier()
        ob = tid * _OROWS
        pltpu.sync_copy(acc.at[pl.ds(ob, _OROWS)], o_hbm.at[pl.ds(ob, _OROWS)])
        tail = _NTILES * _OROWS

        @pl.when(tid == 0)
        def _():
            pltpu.sync_copy(acc.at[pl.ds(tail, _NN - tail)],
                            o_hbm.at[pl.ds(tail, _NN - tail)])

    @pl.when(cid == 0)
    def _():
        _run(ma_hbm, oa_hbm)

    @pl.when(cid == 1)
    def _():
        _run(mb_hbm, ob_hbm)


def _sc_edge_agg(src3, dst3, ma, mb):
    mesh = plsc.VectorSubcoreMesh(core_axis_name="c", subcore_axis_name="s")
    kern = pl.kernel(
        _sc_body,
        out_type=(jax.ShapeDtypeStruct((_NN, _HH), jnp.float32),
                  jax.ShapeDtypeStruct((_NN, _HH), jnp.float32)),
        mesh=mesh,
        compiler_params=pltpu.CompilerParams(use_tc_tiling_on_sc=False),
        scratch_types=[
            pltpu.VMEM_SHARED((_ACC_ROWS, _HH), jnp.float32),
            pltpu.VMEM((_CPG, _CHUNK), jnp.int32),
            pltpu.VMEM((_CPG, _CHUNK), jnp.int32),
            pltpu.VMEM((_CHUNK, _HH), jnp.float32),
            pltpu.VMEM((_CHUNK, _HH), jnp.float32),
            pltpu.SemaphoreType.DMA,
            pltpu.SemaphoreType.DMA,
        ],
    )
    return kern(src3, dst3, ma, mb)


def kernel(x, edge_index, graph_ids, W_lift, b_lift, W_ro, b_ro,
           W_msg0, b_msg0, W_out0, b_out0,
           W_msg1, b_msg1, W_out1, b_out1,
           W_msg2, b_msg2, W_out2, b_out2):
    f32 = jnp.float32
    dH = _HP - _H

    xp = jnp.pad(x, ((0, 0), (0, _RAWP - _RAW)))
    wl = jnp.pad(W_lift, ((0, _RAWP - _RAW), (0, dH)))
    bl = jnp.pad(b_lift, (0, dH)).reshape(1, _HP)
    wm = [jnp.pad(w, ((0, dH), (0, dH))) for w in (W_msg0, W_msg1, W_msg2)]
    bm = [jnp.pad(b, (0, dH)).reshape(1, _HP) for b in (b_msg0, b_msg1, b_msg2)]
    wo = [jnp.pad(w, ((0, dH), (0, dH))) for w in (W_out0, W_out1, W_out2)]
    bo = [jnp.pad(b, (0, dH)).reshape(1, _HP) for b in (b_out0, b_out1, b_out2)]
    wro = jnp.zeros((_HP, 128), f32)
    wro = wro.at[:_H, :_C].set(W_ro)
    wro = wro.at[_HP - 1, :_C].set(b_ro)

    src = edge_index[0]
    dst = edge_index[1]
    pad_e = _EPAD - _EE
    srcp = jnp.concatenate(
        [src, jnp.zeros((pad_e,), jnp.int32)]).reshape(
            _NTILES, _NGRP, _CPG, _CHUNK)
    dstp = jnp.concatenate(
        [dst, jnp.full((pad_e,), _NN, jnp.int32)]).reshape(
            _NTILES, _NGRP, _CPG, _CHUNK)
    gid3 = graph_ids.reshape(_GRID, 1, _BLK)

    ma, mb = _lift_msg(xp, wl, bl, wm[0], bm[0])
    aa, ab = _sc_edge_agg(srcp, dstp, ma, mb)
    for layer in range(2):
        ma, mb = _out_msg(aa, ab, wo[layer], bo[layer],
                          wm[layer + 1], bm[layer + 1])
        aa, ab = _sc_edge_agg(srcp, dstp, ma, mb)
    out = _out_readout(aa, ab, wo[2], bo[2], gid3, wro)
    return out[:_B, :_C]


# speedup vs baseline: 4.0977x; 1.2252x over previous
import functools

import jax
import jax.numpy as jnp
from jax import lax
from jax.experimental import pallas as pl
from jax.experimental.pallas import tpu as pltpu
from jax.experimental.pallas import tpu_sc as plsc

_NN = 10000
_EE = 160000
_RAW = 119
_RAWP = 128
_H = 300
_HP = 320
_HH = 160
_B = 10
_C = 2

_BLK = 2000
_GRID = _NN // _BLK

_NTILES = 16
_CHUNK = 128
_CPG = 16
_NGRP = 5
_EPT = _CHUNK * _CPG * _NGRP
_EPAD = _EPT * _NTILES
_ACC_ROWS = 10112
_ZROWS = _ACC_ROWS // _NTILES
_OROWS = 624


def _lift_msg_body(x_ref, wl_ref, bl_ref, wm_ref, bm_ref, ma_ref, mb_ref):
    h = jnp.dot(x_ref[...], wl_ref[...], preferred_element_type=jnp.float32)
    h = h + bl_ref[...]
    m = jnp.dot(h, wm_ref[...], preferred_element_type=jnp.float32) + bm_ref[...]
    m = jnp.maximum(m, 0.0).astype(jnp.bfloat16)
    ma_ref[...] = m[:, :_HH]
    mb_ref[...] = m[:, _HH:]


def _lift_msg(xp, wl, bl, wm, bm):
    return pl.pallas_call(
        _lift_msg_body,
        grid=(_GRID,),
        in_specs=[
            pl.BlockSpec((_BLK, _RAWP), lambda i: (i, 0)),
            pl.BlockSpec((_RAWP, _HP), lambda i: (0, 0)),
            pl.BlockSpec((1, _HP), lambda i: (0, 0)),
            pl.BlockSpec((_HP, _HP), lambda i: (0, 0)),
            pl.BlockSpec((1, _HP), lambda i: (0, 0)),
        ],
        out_specs=[pl.BlockSpec((_BLK, _HH), lambda i: (i, 0)),
                   pl.BlockSpec((_BLK, _HH), lambda i: (i, 0))],
        out_shape=[jax.ShapeDtypeStruct((_NN, _HH), jnp.bfloat16),
                   jax.ShapeDtypeStruct((_NN, _HH), jnp.bfloat16)],
    )(xp, wl, bl, wm, bm)


def _hidden(aa_ref, ab_ref, wo_ref, bo_ref):
    aa = aa_ref[...].astype(jnp.float32)
    ab = ab_ref[...].astype(jnp.float32)
    h = jnp.dot(aa, wo_ref[0:_HH, :], preferred_element_type=jnp.float32)
    h = h + jnp.dot(ab, wo_ref[_HH:_HP, :], preferred_element_type=jnp.float32)
    return jnp.maximum(h + bo_ref[...], 0.0)


def _out_msg_body(aa_ref, ab_ref, wo_ref, bo_ref, wm_ref, bm_ref, ma_ref, mb_ref):
    h = _hidden(aa_ref, ab_ref, wo_ref, bo_ref)
    m = jnp.dot(h, wm_ref[...], preferred_element_type=jnp.float32) + bm_ref[...]
    m = jnp.maximum(m, 0.0).astype(jnp.bfloat16)
    ma_ref[...] = m[:, :_HH]
    mb_ref[...] = m[:, _HH:]


def _out_msg(aa, ab, wo, bo, wm, bm):
    return pl.pallas_call(
        _out_msg_body,
        grid=(_GRID,),
        in_specs=[
            pl.BlockSpec((_BLK, _HH), lambda i: (i, 0)),
            pl.BlockSpec((_BLK, _HH), lambda i: (i, 0)),
            pl.BlockSpec((_HP, _HP), lambda i: (0, 0)),
            pl.BlockSpec((1, _HP), lambda i: (0, 0)),
            pl.BlockSpec((_HP, _HP), lambda i: (0, 0)),
            pl.BlockSpec((1, _HP), lambda i: (0, 0)),
        ],
        out_specs=[pl.BlockSpec((_BLK, _HH), lambda i: (i, 0)),
                   pl.BlockSpec((_BLK, _HH), lambda i: (i, 0))],
        out_shape=[jax.ShapeDtypeStruct((_NN, _HH), jnp.bfloat16),
                   jax.ShapeDtypeStruct((_NN, _HH), jnp.bfloat16)],
    )(aa, ab, wo, bo, wm, bm)


def _out_readout_body(aa_ref, ab_ref, wo_ref, bo_ref, gid_ref, wro_ref,
                      out_ref, acc_ref):
    i = pl.program_id(0)

    @pl.when(i == 0)
    def _():
        acc_ref[...] = jnp.zeros_like(acc_ref)

    h = _hidden(aa_ref, ab_ref, wo_ref, bo_ref)
    col = lax.broadcasted_iota(jnp.int32, (_BLK, _HP), 1)
    h = jnp.where(col == _HP - 1, 1.0, h)
    gid = gid_ref[0, 0, :]
    onehot = (lax.broadcasted_iota(jnp.int32, (16, _BLK), 0)
              == gid[None, :]).astype(jnp.float32)
    acc_ref[...] += jnp.dot(onehot, h, preferred_element_type=jnp.float32)

    @pl.when(i == _GRID - 1)
    def _():
        out_ref[...] = jnp.dot(acc_ref[...], wro_ref[...],
                               preferred_element_type=jnp.float32)


def _out_readout(aa, ab, wo, bo, gid3, wro):
    return pl.pallas_call(
        _out_readout_body,
        grid=(_GRID,),
        in_specs=[
            pl.BlockSpec((_BLK, _HH), lambda i: (i, 0)),
            pl.BlockSpec((_BLK, _HH), lambda i: (i, 0)),
            pl.BlockSpec((_HP, _HP), lambda i: (0, 0)),
            pl.BlockSpec((1, _HP), lambda i: (0, 0)),
            pl.BlockSpec((1, 1, _BLK), lambda i: (i, 0, 0)),
            pl.BlockSpec((_HP, 128), lambda i: (0, 0)),
        ],
        out_specs=pl.BlockSpec((16, 128), lambda i: (0, 0)),
        out_shape=jax.ShapeDtypeStruct((16, 128), jnp.float32),
        scratch_shapes=[pltpu.VMEM((16, _HP), jnp.float32)],
    )(aa, ab, wo, bo, gid3, wro)


def _sc_body(src_hbm, dst_hbm, ma_hbm, mb_hbm, oa_hbm, ob_hbm,
             acc, sidx, didx, b0, b1, b2, b3,
             g0, g1, g2, g3, t0, t1, t2, t3):
    cid = lax.axis_index("c")
    tid = lax.axis_index("s")
    bufs = (b0, b1, b2, b3)
    gsem = (g0, g1, g2, g3)
    ssem = (t0, t1, t2, t3)

    zero = jnp.zeros((32,), jnp.bfloat16)

    def _zrow(i, carry):
        for j in range(_HH // 32):
            b0[i, pl.ds(j * 32, 32)] = zero
        return carry

    lax.fori_loop(0, _CHUNK, _zrow, 0)
    zb = tid * _ZROWS
    for q in range(_ZROWS // _CHUNK):
        pltpu.sync_copy(b0, acc.at[pl.ds(zb + q * _CHUNK, _CHUNK)])
    rem = _ZROWS % _CHUNK
    pltpu.sync_copy(b0.at[pl.ds(0, rem)],
                    acc.at[pl.ds(zb + _ZROWS - rem, rem)])
    plsc.subcore_barrier()

    def _run(m_hbm, o_hbm):
        def _wait_scatter(b, k):
            pltpu.make_async_copy(bufs[b], acc.at[didx.at[k]], ssem[b]).wait()

        def _drain_tail():
            for b in range(4):
                _wait_scatter(b, _CPG - 4 + b)

        def _group(g, carry):
            @pl.when(g > 0)
            def _():
                _drain_tail()

            pltpu.sync_copy(src_hbm.at[tid, g], sidx)
            pltpu.sync_copy(dst_hbm.at[tid, g], didx)
            for b in range(4):
                pltpu.async_copy(m_hbm.at[sidx.at[b]], bufs[b], gsem[b])
            for k in range(_CPG):
                b = k % 4
                pltpu.make_async_copy(m_hbm.at[sidx.at[k]], bufs[b],
                                      gsem[b]).wait()
                pltpu.async_copy(bufs[b], acc.at[didx.at[k]], ssem[b],
                                 add=True)
                if k >= 2 and k + 2 < _CPG:
                    b2 = (k - 2) % 4
                    _wait_scatter(b2, k - 2)
                    pltpu.async_copy(m_hbm.at[sidx.at[k + 2]], bufs[b2],
                                     gsem[b2])
            return carry

        lax.fori_loop(0, _NGRP, _group, 0)
        _drain_tail()
        plsc.subcore_barrier()
        ob = tid * _OROWS
        pltpu.sync_copy(acc.at[pl.ds(ob, _OROWS)], o_hbm.at[pl.ds(ob, _OROWS)])
        tail = _NTILES * _OROWS

        @pl.when(tid == 0)
        def _():
            pltpu.sync_copy(acc.at[pl.ds(tail, _NN - tail)],
                            o_hbm.at[pl.ds(tail, _NN - tail)])

    @pl.when(cid == 0)
    def _():
        _run(ma_hbm, oa_hbm)

    @pl.when(cid == 1)
    def _():
        _run(mb_hbm, ob_hbm)


def _sc_edge_agg(src3, dst3, ma, mb):
    mesh = plsc.VectorSubcoreMesh(core_axis_name="c", subcore_axis_name="s")
    kern = pl.kernel(
        _sc_body,
        out_type=(jax.ShapeDtypeStruct((_NN, _HH), jnp.bfloat16),
                  jax.ShapeDtypeStruct((_NN, _HH), jnp.bfloat16)),
        mesh=mesh,
        compiler_params=pltpu.CompilerParams(use_tc_tiling_on_sc=False),
        scratch_types=[
            pltpu.VMEM_SHARED((_ACC_ROWS, _HH), jnp.bfloat16),
            pltpu.VMEM((_CPG, _CHUNK), jnp.int32),
            pltpu.VMEM((_CPG, _CHUNK), jnp.int32),
            pltpu.VMEM((_CHUNK, _HH), jnp.bfloat16),
            pltpu.VMEM((_CHUNK, _HH), jnp.bfloat16),
            pltpu.VMEM((_CHUNK, _HH), jnp.bfloat16),
            pltpu.VMEM((_CHUNK, _HH), jnp.bfloat16),
            pltpu.SemaphoreType.DMA,
            pltpu.SemaphoreType.DMA,
            pltpu.SemaphoreType.DMA,
            pltpu.SemaphoreType.DMA,
            pltpu.SemaphoreType.DMA,
            pltpu.SemaphoreType.DMA,
            pltpu.SemaphoreType.DMA,
            pltpu.SemaphoreType.DMA,
        ],
    )
    return kern(src3, dst3, ma, mb)


def kernel(x, edge_index, graph_ids, W_lift, b_lift, W_ro, b_ro,
           W_msg0, b_msg0, W_out0, b_out0,
           W_msg1, b_msg1, W_out1, b_out1,
           W_msg2, b_msg2, W_out2, b_out2):
    f32 = jnp.float32
    dH = _HP - _H

    xp = jnp.pad(x, ((0, 0), (0, _RAWP - _RAW)))
    wl = jnp.pad(W_lift, ((0, _RAWP - _RAW), (0, dH)))
    bl = jnp.pad(b_lift, (0, dH)).reshape(1, _HP)
    wm = [jnp.pad(w, ((0, dH), (0, dH))) for w in (W_msg0, W_msg1, W_msg2)]
    bm = [jnp.pad(b, (0, dH)).reshape(1, _HP) for b in (b_msg0, b_msg1, b_msg2)]
    wo = [jnp.pad(w, ((0, dH), (0, dH))) for w in (W_out0, W_out1, W_out2)]
    bo = [jnp.pad(b, (0, dH)).reshape(1, _HP) for b in (b_out0, b_out1, b_out2)]
    wro = jnp.zeros((_HP, 128), f32)
    wro = wro.at[:_H, :_C].set(W_ro)
    wro = wro.at[_HP - 1, :_C].set(b_ro)

    src = edge_index[0]
    dst = edge_index[1]
    pad_e = _EPAD - _EE
    srcp = jnp.concatenate(
        [src, jnp.zeros((pad_e,), jnp.int32)]).reshape(
            _NTILES, _NGRP, _CPG, _CHUNK)
    dstp = jnp.concatenate(
        [dst, jnp.full((pad_e,), _NN, jnp.int32)]).reshape(
            _NTILES, _NGRP, _CPG, _CHUNK)
    gid3 = graph_ids.reshape(_GRID, 1, _BLK)

    ma, mb = _lift_msg(xp, wl, bl, wm[0], bm[0])
    aa, ab = _sc_edge_agg(srcp, dstp, ma, mb)
    for layer in range(2):
        ma, mb = _out_msg(aa, ab, wo[layer], bo[layer],
                          wm[layer + 1], bm[layer + 1])
        aa, ab = _sc_edge_agg(srcp, dstp, ma, mb)
    out = _out_readout(aa, ab, wo[2], bo[2], gid3, wro)
    return out[:_B, :_C]


# speedup vs baseline: 4.2978x; 1.0488x over previous
import functools

import jax
import jax.numpy as jnp
from jax import lax
from jax.experimental import pallas as pl
from jax.experimental.pallas import tpu as pltpu
from jax.experimental.pallas import tpu_sc as plsc

_NN = 10000
_EE = 160000
_RAW = 119
_RAWP = 128
_H = 300
_HP = 320
_HH = 160
_B = 10
_C = 2

_BLK = 2000
_GRID = _NN // _BLK

_NTILES = 16
_CHUNK = 128
_CPG = 20
_NGRP = 4
_EPT = _CHUNK * _CPG * _NGRP
_EPAD = _EPT * _NTILES
_ACC_ROWS = 10112
_ZROWS = _ACC_ROWS // _NTILES
_OROWS = 624


def _lift_msg_body(x_ref, wl_ref, bl_ref, wm_ref, bm_ref, ma_ref, mb_ref):
    h = jnp.dot(x_ref[...], wl_ref[...], preferred_element_type=jnp.float32)
    h = h + bl_ref[...]
    m = jnp.dot(h, wm_ref[...], preferred_element_type=jnp.float32) + bm_ref[...]
    m = jnp.maximum(m, 0.0).astype(jnp.bfloat16)
    ma_ref[...] = m[:, :_HH]
    mb_ref[...] = m[:, _HH:]


def _lift_msg(xp, wl, bl, wm, bm):
    return pl.pallas_call(
        _lift_msg_body,
        grid=(_GRID,),
        in_specs=[
            pl.BlockSpec((_BLK, _RAWP), lambda i: (i, 0)),
            pl.BlockSpec((_RAWP, _HP), lambda i: (0, 0)),
            pl.BlockSpec((1, _HP), lambda i: (0, 0)),
            pl.BlockSpec((_HP, _HP), lambda i: (0, 0)),
            pl.BlockSpec((1, _HP), lambda i: (0, 0)),
        ],
        out_specs=[pl.BlockSpec((_BLK, _HH), lambda i: (i, 0)),
                   pl.BlockSpec((_BLK, _HH), lambda i: (i, 0))],
        out_shape=[jax.ShapeDtypeStruct((_NN, _HH), jnp.bfloat16),
                   jax.ShapeDtypeStruct((_NN, _HH), jnp.bfloat16)],
    )(xp, wl, bl, wm, bm)


def _hidden(aa_ref, ab_ref, wo_ref, bo_ref):
    aa = aa_ref[...].astype(jnp.float32)
    ab = ab_ref[...].astype(jnp.float32)
    h = jnp.dot(aa, wo_ref[0:_HH, :], preferred_element_type=jnp.float32)
    h = h + jnp.dot(ab, wo_ref[_HH:_HP, :], preferred_element_type=jnp.float32)
    return jnp.maximum(h + bo_ref[...], 0.0)


def _out_msg_body(aa_ref, ab_ref, wo_ref, bo_ref, wm_ref, bm_ref, ma_ref, mb_ref):
    h = _hidden(aa_ref, ab_ref, wo_ref, bo_ref)
    m = jnp.dot(h, wm_ref[...], preferred_element_type=jnp.float32) + bm_ref[...]
    m = jnp.maximum(m, 0.0).astype(jnp.bfloat16)
    ma_ref[...] = m[:, :_HH]
    mb_ref[...] = m[:, _HH:]


def _out_msg(aa, ab, wo, bo, wm, bm):
    return pl.pallas_call(
        _out_msg_body,
        grid=(_GRID,),
        in_specs=[
            pl.BlockSpec((_BLK, _HH), lambda i: (i, 0)),
            pl.BlockSpec((_BLK, _HH), lambda i: (i, 0)),
            pl.BlockSpec((_HP, _HP), lambda i: (0, 0)),
            pl.BlockSpec((1, _HP), lambda i: (0, 0)),
            pl.BlockSpec((_HP, _HP), lambda i: (0, 0)),
            pl.BlockSpec((1, _HP), lambda i: (0, 0)),
        ],
        out_specs=[pl.BlockSpec((_BLK, _HH), lambda i: (i, 0)),
                   pl.BlockSpec((_BLK, _HH), lambda i: (i, 0))],
        out_shape=[jax.ShapeDtypeStruct((_NN, _HH), jnp.bfloat16),
                   jax.ShapeDtypeStruct((_NN, _HH), jnp.bfloat16)],
    )(aa, ab, wo, bo, wm, bm)


def _out_readout_body(aa_ref, ab_ref, wo_ref, bo_ref, gid_ref, wro_ref,
                      out_ref, acc_ref):
    i = pl.program_id(0)

    @pl.when(i == 0)
    def _():
        acc_ref[...] = jnp.zeros_like(acc_ref)

    h = _hidden(aa_ref, ab_ref, wo_ref, bo_ref)
    col = lax.broadcasted_iota(jnp.int32, (_BLK, _HP), 1)
    h = jnp.where(col == _HP - 1, 1.0, h)
    gid = gid_ref[0, 0, :]
    onehot = (lax.broadcasted_iota(jnp.int32, (16, _BLK), 0)
              == gid[None, :]).astype(jnp.float32)
    acc_ref[...] += jnp.dot(onehot, h, preferred_element_type=jnp.float32)

    @pl.when(i == _GRID - 1)
    def _():
        out_ref[...] = jnp.dot(acc_ref[...], wro_ref[...],
                               preferred_element_type=jnp.float32)


def _out_readout(aa, ab, wo, bo, gid3, wro):
    return pl.pallas_call(
        _out_readout_body,
        grid=(_GRID,),
        in_specs=[
            pl.BlockSpec((_BLK, _HH), lambda i: (i, 0)),
            pl.BlockSpec((_BLK, _HH), lambda i: (i, 0)),
            pl.BlockSpec((_HP, _HP), lambda i: (0, 0)),
            pl.BlockSpec((1, _HP), lambda i: (0, 0)),
            pl.BlockSpec((1, 1, _BLK), lambda i: (i, 0, 0)),
            pl.BlockSpec((_HP, 128), lambda i: (0, 0)),
        ],
        out_specs=pl.BlockSpec((16, 128), lambda i: (0, 0)),
        out_shape=jax.ShapeDtypeStruct((16, 128), jnp.float32),
        scratch_shapes=[pltpu.VMEM((16, _HP), jnp.float32)],
    )(aa, ab, wo, bo, gid3, wro)


def _sc_body(src_hbm, dst_hbm, ma_hbm, mb_hbm, oa_hbm, ob_hbm,
             acc, sidx, didx, b0, b1, b2, b3, b4,
             g0, g1, g2, g3, g4, t0, t1, t2, t3, t4):
    cid = lax.axis_index("c")
    tid = lax.axis_index("s")
    bufs = (b0, b1, b2, b3, b4)
    gsem = (g0, g1, g2, g3, g4)
    ssem = (t0, t1, t2, t3, t4)

    zero = jnp.zeros((32,), jnp.bfloat16)

    def _zrow(i, carry):
        for j in range(_HH // 32):
            b0[i, pl.ds(j * 32, 32)] = zero
        return carry

    lax.fori_loop(0, _CHUNK, _zrow, 0)
    zb = tid * _ZROWS
    for q in range(_ZROWS // _CHUNK):
        pltpu.sync_copy(b0, acc.at[pl.ds(zb + q * _CHUNK, _CHUNK)])
    rem = _ZROWS % _CHUNK
    pltpu.sync_copy(b0.at[pl.ds(0, rem)],
                    acc.at[pl.ds(zb + _ZROWS - rem, rem)])
    plsc.subcore_barrier()

    def _run(m_hbm, o_hbm):
        def _wait_scatter(b, k):
            pltpu.make_async_copy(bufs[b], acc.at[didx.at[k]], ssem[b]).wait()

        def _drain_tail():
            for b in range(5):
                _wait_scatter(b, _CPG - 5 + b)

        def _group(g, carry):
            @pl.when(g > 0)
            def _():
                _drain_tail()

            pltpu.sync_copy(src_hbm.at[tid, g], sidx)
            pltpu.sync_copy(dst_hbm.at[tid, g], didx)
            for b in range(5):
                pltpu.async_copy(m_hbm.at[sidx.at[b]], bufs[b], gsem[b])
            for k in range(_CPG):
                b = k % 5
                pltpu.make_async_copy(m_hbm.at[sidx.at[k]], bufs[b],
                                      gsem[b]).wait()
                pltpu.async_copy(bufs[b], acc.at[didx.at[k]], ssem[b],
                                 add=True)
                if k >= 2 and k + 3 < _CPG:
                    b2 = (k - 2) % 5
                    _wait_scatter(b2, k - 2)
                    pltpu.async_copy(m_hbm.at[sidx.at[k + 3]], bufs[b2],
                                     gsem[b2])
            return carry

        lax.fori_loop(0, _NGRP, _group, 0)
        _drain_tail()
        plsc.subcore_barrier()
        ob = tid * _OROWS
        pltpu.sync_copy(acc.at[pl.ds(ob, _OROWS)], o_hbm.at[pl.ds(ob, _OROWS)])
        tail = _NTILES * _OROWS

        @pl.when(tid == 0)
        def _():
            pltpu.sync_copy(acc.at[pl.ds(tail, _NN - tail)],
                            o_hbm.at[pl.ds(tail, _NN - tail)])

    @pl.when(cid == 0)
    def _():
        _run(ma_hbm, oa_hbm)

    @pl.when(cid == 1)
    def _():
        _run(mb_hbm, ob_hbm)


def _sc_edge_agg(src3, dst3, ma, mb):
    mesh = plsc.VectorSubcoreMesh(core_axis_name="c", subcore_axis_name="s")
    kern = pl.kernel(
        _sc_body,
        out_type=(jax.ShapeDtypeStruct((_NN, _HH), jnp.bfloat16),
                  jax.ShapeDtypeStruct((_NN, _HH), jnp.bfloat16)),
        mesh=mesh,
        compiler_params=pltpu.CompilerParams(use_tc_tiling_on_sc=False),
        scratch_types=[
            pltpu.VMEM_SHARED((_ACC_ROWS, _HH), jnp.bfloat16),
            pltpu.VMEM((_CPG, _CHUNK), jnp.int32),
            pltpu.VMEM((_CPG, _CHUNK), jnp.int32),
            pltpu.VMEM((_CHUNK, _HH), jnp.bfloat16),
            pltpu.VMEM((_CHUNK, _HH), jnp.bfloat16),
            pltpu.VMEM((_CHUNK, _HH), jnp.bfloat16),
            pltpu.VMEM((_CHUNK, _HH), jnp.bfloat16),
            pltpu.VMEM((_CHUNK, _HH), jnp.bfloat16),
        ] + [pltpu.SemaphoreType.DMA] * 10,
    )
    return kern(src3, dst3, ma, mb)


def kernel(x, edge_index, graph_ids, W_lift, b_lift, W_ro, b_ro,
           W_msg0, b_msg0, W_out0, b_out0,
           W_msg1, b_msg1, W_out1, b_out1,
           W_msg2, b_msg2, W_out2, b_out2):
    f32 = jnp.float32
    dH = _HP - _H

    xp = jnp.pad(x, ((0, 0), (0, _RAWP - _RAW)))
    wl = jnp.pad(W_lift, ((0, _RAWP - _RAW), (0, dH)))
    bl = jnp.pad(b_lift, (0, dH)).reshape(1, _HP)
    wm = [jnp.pad(w, ((0, dH), (0, dH))) for w in (W_msg0, W_msg1, W_msg2)]
    bm = [jnp.pad(b, (0, dH)).reshape(1, _HP) for b in (b_msg0, b_msg1, b_msg2)]
    wo = [jnp.pad(w, ((0, dH), (0, dH))) for w in (W_out0, W_out1, W_out2)]
    bo = [jnp.pad(b, (0, dH)).reshape(1, _HP) for b in (b_out0, b_out1, b_out2)]
    wro = jnp.zeros((_HP, 128), f32)
    wro = wro.at[:_H, :_C].set(W_ro)
    wro = wro.at[_HP - 1, :_C].set(b_ro)

    src = edge_index[0]
    dst = edge_index[1]
    pad_e = _EPAD - _EE
    srcp = jnp.concatenate(
        [src, jnp.zeros((pad_e,), jnp.int32)]).reshape(
            _NTILES, _NGRP, _CPG, _CHUNK)
    dstp = jnp.concatenate(
        [dst, jnp.full((pad_e,), _NN, jnp.int32)]).reshape(
            _NTILES, _NGRP, _CPG, _CHUNK)
    gid3 = graph_ids.reshape(_GRID, 1, _BLK)

    ma, mb = _lift_msg(xp, wl, bl, wm[0], bm[0])
    aa, ab = _sc_edge_agg(srcp, dstp, ma, mb)
    for layer in range(2):
        ma, mb = _out_msg(aa, ab, wo[layer], bo[layer],
                          wm[layer + 1], bm[layer + 1])
        aa, ab = _sc_edge_agg(srcp, dstp, ma, mb)
    out = _out_readout(aa, ab, wo[2], bo[2], gid3, wro)
    return out[:_B, :_C]
